# baseline (speedup 1.0000x reference)
import functools

import jax
import jax.numpy as jnp
from jax import lax
from jax.experimental import pallas as pl
from jax.experimental.pallas import tpu as pltpu
from jax.experimental.pallas import tpu_sc as plsc

_B, _D, _H, _W, _K = 4, 16, 512, 512, 8
_DELTA_VAR = 1.0
_DELTA_DIST = 2.0
_SQRT_D = 4.0

_NW = 32
_RPW = _H // _NW
_CR = 4
_NCH = _RPW // _CR
_SEG = _K * (_D + 1)
_ACC = _B * _SEG
_GPC = _CR * _W // 16


def _seg_body(data_hbm, lab_hbm, out_hbm, lab_v, x_v, acc2, accf,
              sem0, sem1):
    wid = lax.axis_index("s") * 2 + lax.axis_index("c")
    row0 = wid * _RPW
    f32 = jnp.float32
    sems = (sem0, sem1)

    @plsc.parallel_loop(0, 16 * _ACC // 16, unroll=8)
    def _zero(i):
        acc2[pl.ds(i * 16, 16)] = jnp.zeros((16,), f32)

    def issue(n, buf):
        b = n // _NCH
        r = row0 + (n % _NCH) * _CR
        for d in range(_D):
            pltpu.async_copy(
                data_hbm.at[b, d, pl.ds(r, _CR), :], x_v.at[buf, d], sems[buf])
        pltpu.async_copy(
            lab_hbm.at[b, pl.ds(r, _CR), :], lab_v.at[buf], sems[buf])

    def wait_buf(buf):
        pltpu.make_async_copy(
            data_hbm.at[0, :, pl.ds(0, _CR), :], x_v.at[buf], sems[buf]).wait()
        pltpu.make_async_copy(
            lab_hbm.at[0, pl.ds(0, _CR), :], lab_v.at[buf], sems[buf]).wait()

    ntot = _B * _NCH
    ones16 = jnp.ones((16,), f32)
    laneoff = lax.iota(jnp.int32, 16)

    issue(0, 0)

    def outer(m, _):
        for buf in range(2):
            n = m * 2 + buf
            wait_buf(buf)

            @pl.when(n + 1 < ntot)
            def _prefetch():
                issue(n + 1, buf ^ 1)

            b_seg16 = (n // _NCH) * (_SEG * 16)

            @plsc.parallel_loop(0, _GPC, unroll=4)
            def _scat(g):
                r = g // (_W // 16)
                sl = pl.ds((g % (_W // 16)) * 16, 16)
                ix = (lab_v[buf, r, sl] * ((_D + 1) * 16)
                      + b_seg16 + laneoff)
                for d in range(_D):
                    plsc.addupdate_scatter(
                        acc2, [ix + d * 16], x_v[buf, d, r, sl])
                plsc.addupdate_scatter(acc2, [ix + _D * 16], ones16)

        return 0

    lax.fori_loop(0, ntot // 2, outer, 0)

    @plsc.parallel_loop(0, _ACC // 16, unroll=2)
    def _fold(j):
        s = jnp.zeros((16,), f32)
        base = j * 256 + lax.iota(jnp.int32, 16) * 16
        for m in range(16):
            s = s + plsc.load_gather(acc2, [base + m])
        accf[pl.ds(j * 16, 16)] = s

    pltpu.sync_copy(accf, out_hbm.at[wid])


@functools.lru_cache(maxsize=1)
def _seg_sums_kernel():
    return pl.kernel(
        _seg_body,
        mesh=plsc.VectorSubcoreMesh(core_axis_name="c", subcore_axis_name="s"),
        compiler_params=pltpu.CompilerParams(needs_layout_passes=False),
        out_type=jax.ShapeDtypeStruct((_NW, _ACC), jnp.float32),
        scratch_types=[
            pltpu.VMEM((2, _CR, _W), jnp.int32),
            pltpu.VMEM((2, _D, _CR, _W), jnp.float32),
            pltpu.VMEM((16 * _ACC,), jnp.float32),
            pltpu.VMEM((_ACC,), jnp.float32),
            pltpu.SemaphoreType.DMA,
            pltpu.SemaphoreType.DMA,
        ],
    )


def _seg_sums(data, labels):
    return _seg_sums_kernel()(data, labels)


_BH = 128
_NJ = _H // _BH


def _loss_body(data_ref, lab_ref, part_ref, out_ref):
    b = pl.program_id(0)
    j = pl.program_id(1)
    f32 = jnp.float32

    @pl.when((b == 0) & (j == 0))
    def _init():
        out_ref[...] = jnp.zeros((1, 1), f32)

    pm = jnp.sum(part_ref[...], axis=(0, 1))
    sums_t = pm[:, :_D]
    counts = pm[:, _D:_D + 1]
    centers_t = sums_t / jnp.maximum(counts, 1.0)
    present = counts > 0.0
    n_c = jnp.sum(present.astype(f32))
    cn2 = jnp.sum(centers_t * centers_t, axis=1, keepdims=True)
    ones_11 = jnp.ones((1, 1), f32)
    cn2_row = lax.dot_general(
        ones_11, cn2, (((1,), (1,)), ((), ())), preferred_element_type=f32)
    ones_1d = jnp.ones((1, _D), f32)
    dn_d = (((1,), (0,)), ((), ()))

    x = data_ref[0]
    lab = lab_ref[0]
    ks = lax.broadcasted_iota(jnp.int32, (_K, _BH, _W), 0)
    oh = (lab[None] == ks).astype(f32)
    s3 = lax.dot_general(
        ones_1d, x * x, dn_d, preferred_element_type=f32)
    proj = lax.dot_general(
        centers_t, x, dn_d, preferred_element_type=f32)
    cnl = lax.dot_general(
        cn2_row, oh, dn_d, preferred_element_type=f32)
    t = jnp.sum(oh * proj, axis=0)
    d2 = s3[0] + cnl[0] - 2.0 * t
    dd = jnp.sqrt(jnp.maximum(d2, 0.0))
    h = jnp.maximum(dd - _DELTA_VAR, 0.0)
    var_sum = jnp.sum(h * h)

    total = jnp.where(n_c > 1.0, var_sum / jnp.maximum(n_c, 1.0), 0.0)

    @pl.when(j == 0)
    def _tiny_terms():
        g = lax.dot_general(
            centers_t, centers_t, (((1,), (1,)), ((), ())),
            preferred_element_type=f32)
        counts_row = lax.dot_general(
            ones_11, counts, (((1,), (1,)), ((), ())),
            preferred_element_type=f32)
        sq_c = cn2 + cn2_row - 2.0 * g
        ri = lax.broadcasted_iota(jnp.int32, (_K, _K), 0)
        ci = lax.broadcasted_iota(jnp.int32, (_K, _K), 1)
        pair_mask = (ri < ci) & present & (counts_row > 0.0)
        dist = jnp.sqrt(jnp.where(pair_mask, sq_c, 1.0))
        hc = jnp.maximum(2.0 * _DELTA_DIST - dist, 0.0) ** 2
        dist_sum = jnp.sum(jnp.where(pair_mask, hc, 0.0))
        dist_term = dist_sum / jnp.maximum(n_c * (n_c - 1.0), 1.0)

        reg_mask = present & (cn2 > 0.0)
        cn = jnp.sqrt(jnp.where(reg_mask, cn2, 1.0))
        reg_vals = jnp.maximum(cn - _SQRT_D, 0.0)
        reg_term = (jnp.sum(jnp.where(reg_mask, reg_vals, 0.0))
                    / jnp.maximum(n_c, 1.0))
        out_ref[...] += jnp.full(
            (1, 1), jnp.where(n_c > 1.0, dist_term + reg_term, 0.0))

    out_ref[...] += jnp.full((1, 1), total)


def kernel(data, labels):
    labels = labels.astype(jnp.int32)
    partials = _seg_sums(data, labels)
    partials4 = partials.reshape(_NW, _B, _K, _D + 1)
    out = pl.pallas_call(
        _loss_body,
        grid=(_B, _NJ),
        in_specs=[
            pl.BlockSpec((1, _D, _BH, _W), lambda b, j: (b, 0, j, 0)),
            pl.BlockSpec((1, _BH, _W), lambda b, j: (b, j, 0)),
            pl.BlockSpec((_NW, 1, _K, _D + 1), lambda b, j: (0, b, 0, 0)),
        ],
        out_specs=pl.BlockSpec((1, 1), lambda b, j: (0, 0)),
        out_shape=jax.ShapeDtypeStruct((1, 1), jnp.float32),
    )(data, labels, partials4)
    return out[0, 0] / jnp.float32(_B)


# speedup vs baseline: 1.0749x; 1.0749x over previous
import functools

import jax
import jax.numpy as jnp
from jax import lax
from jax.experimental import pallas as pl
from jax.experimental.pallas import tpu as pltpu
from jax.experimental.pallas import tpu_sc as plsc

_B, _D, _H, _W, _K = 4, 16, 512, 512, 8
_DELTA_VAR = 1.0
_DELTA_DIST = 2.0
_SQRT_D = 4.0

_NW = 32
_RPW = _H // _NW
_CR = 4
_NCH = _RPW // _CR
_SEG = _K * (_D + 1)
_ACC = _B * _SEG
_GPC = _CR * _W // 16


def _seg_body(data_hbm, lab_hbm, out_hbm, lab_v, x_v, acc2, accf,
              sem0, sem1):
    wid = lax.axis_index("s") * 2 + lax.axis_index("c")
    row0 = wid * _RPW
    f32 = jnp.float32
    sems = (sem0, sem1)

    @plsc.parallel_loop(0, 16 * _ACC // 16, unroll=8)
    def _zero(i):
        acc2[pl.ds(i * 16, 16)] = jnp.zeros((16,), f32)

    def issue(n, buf):
        b = n // _NCH
        r = row0 + (n % _NCH) * _CR
        for d in range(_D):
            pltpu.async_copy(
                data_hbm.at[b, d, pl.ds(r, _CR), :], x_v.at[buf, d], sems[buf])
        pltpu.async_copy(
            lab_hbm.at[b, pl.ds(r, _CR), :], lab_v.at[buf], sems[buf])

    def wait_buf(buf):
        pltpu.make_async_copy(
            data_hbm.at[0, :, pl.ds(0, _CR), :], x_v.at[buf], sems[buf]).wait()
        pltpu.make_async_copy(
            lab_hbm.at[0, pl.ds(0, _CR), :], lab_v.at[buf], sems[buf]).wait()

    ntot = _B * _NCH
    ones16 = jnp.ones((16,), f32)
    laneoff = lax.iota(jnp.int32, 16)

    issue(0, 0)

    def outer(m, _):
        for buf in range(2):
            n = m * 2 + buf
            wait_buf(buf)

            @pl.when(n + 1 < ntot)
            def _prefetch():
                issue(n + 1, buf ^ 1)

            b_seg16 = (n // _NCH) * (_SEG * 16)

            @plsc.parallel_loop(0, _GPC, unroll=2)
            def _scat(g):
                r = g // (_W // 16)
                sl = pl.ds((g % (_W // 16)) * 16, 16)
                ix = (lab_v[buf, r, sl] * ((_D + 1) * 16)
                      + b_seg16 + laneoff)
                for d in range(_D):
                    plsc.addupdate_scatter(
                        acc2, [ix + d * 16], x_v[buf, d, r, sl])
                plsc.addupdate_scatter(acc2, [ix + _D * 16], ones16)

        return 0

    lax.fori_loop(0, ntot // 2, outer, 0)

    @plsc.parallel_loop(0, _ACC // 16, unroll=2)
    def _fold(j):
        s = jnp.zeros((16,), f32)
        base = j * 256 + lax.iota(jnp.int32, 16) * 16
        for m in range(16):
            s = s + plsc.load_gather(acc2, [base + m])
        accf[pl.ds(j * 16, 16)] = s

    pltpu.sync_copy(accf, out_hbm.at[wid])


@functools.lru_cache(maxsize=1)
def _seg_sums_kernel():
    return pl.kernel(
        _seg_body,
        mesh=plsc.VectorSubcoreMesh(core_axis_name="c", subcore_axis_name="s"),
        compiler_params=pltpu.CompilerParams(needs_layout_passes=False),
        out_type=jax.ShapeDtypeStruct((_NW, _ACC), jnp.float32),
        scratch_types=[
            pltpu.VMEM((2, _CR, _W), jnp.int32),
            pltpu.VMEM((2, _D, _CR, _W), jnp.float32),
            pltpu.VMEM((16 * _ACC,), jnp.float32),
            pltpu.VMEM((_ACC,), jnp.float32),
            pltpu.SemaphoreType.DMA,
            pltpu.SemaphoreType.DMA,
        ],
    )


def _seg_sums(data, labels):
    return _seg_sums_kernel()(data, labels)


_BH = 128
_NJ = _H // _BH


def _loss_body(data_ref, lab_ref, part_ref, out_ref):
    b = pl.program_id(0)
    j = pl.program_id(1)
    f32 = jnp.float32

    @pl.when((b == 0) & (j == 0))
    def _init():
        out_ref[...] = jnp.zeros((1, 1), f32)

    pm = jnp.sum(part_ref[...], axis=(0, 1))
    sums_t = pm[:, :_D]
    counts = pm[:, _D:_D + 1]
    centers_t = sums_t / jnp.maximum(counts, 1.0)
    present = counts > 0.0
    n_c = jnp.sum(present.astype(f32))
    cn2 = jnp.sum(centers_t * centers_t, axis=1, keepdims=True)
    ones_11 = jnp.ones((1, 1), f32)
    cn2_row = lax.dot_general(
        ones_11, cn2, (((1,), (1,)), ((), ())), preferred_element_type=f32)
    ones_1d = jnp.ones((1, _D), f32)
    dn_d = (((1,), (0,)), ((), ()))

    x = data_ref[0]
    lab = lab_ref[0]
    ks = lax.broadcasted_iota(jnp.int32, (_K, _BH, _W), 0)
    oh = (lab[None] == ks).astype(f32)
    s3 = lax.dot_general(
        ones_1d, x * x, dn_d, preferred_element_type=f32)
    proj = lax.dot_general(
        centers_t, x, dn_d, preferred_element_type=f32)
    c0 = (lab & 1) == 1
    c1 = (lab & 2) == 2
    c2 = (lab & 4) == 4
    v0 = jnp.where(c0, cn2[1, 0], cn2[0, 0])
    v1 = jnp.where(c0, cn2[3, 0], cn2[2, 0])
    v2 = jnp.where(c0, cn2[5, 0], cn2[4, 0])
    v3 = jnp.where(c0, cn2[7, 0], cn2[6, 0])
    w0 = jnp.where(c1, v1, v0)
    w1 = jnp.where(c1, v3, v2)
    cnl = jnp.where(c2, w1, w0)
    t = jnp.sum(oh * proj, axis=0)
    d2 = s3[0] + cnl - 2.0 * t
    dd = jnp.sqrt(jnp.maximum(d2, 0.0))
    h = jnp.maximum(dd - _DELTA_VAR, 0.0)
    var_sum = jnp.sum(h * h)

    total = jnp.where(n_c > 1.0, var_sum / jnp.maximum(n_c, 1.0), 0.0)

    @pl.when(j == 0)
    def _tiny_terms():
        g = lax.dot_general(
            centers_t, centers_t, (((1,), (1,)), ((), ())),
            preferred_element_type=f32)
        counts_row = lax.dot_general(
            ones_11, counts, (((1,), (1,)), ((), ())),
            preferred_element_type=f32)
        sq_c = cn2 + cn2_row - 2.0 * g
        ri = lax.broadcasted_iota(jnp.int32, (_K, _K), 0)
        ci = lax.broadcasted_iota(jnp.int32, (_K, _K), 1)
        pair_mask = (ri < ci) & present & (counts_row > 0.0)
        dist = jnp.sqrt(jnp.where(pair_mask, sq_c, 1.0))
        hc = jnp.maximum(2.0 * _DELTA_DIST - dist, 0.0) ** 2
        dist_sum = jnp.sum(jnp.where(pair_mask, hc, 0.0))
        dist_term = dist_sum / jnp.maximum(n_c * (n_c - 1.0), 1.0)

        reg_mask = present & (cn2 > 0.0)
        cn = jnp.sqrt(jnp.where(reg_mask, cn2, 1.0))
        reg_vals = jnp.maximum(cn - _SQRT_D, 0.0)
        reg_term = (jnp.sum(jnp.where(reg_mask, reg_vals, 0.0))
                    / jnp.maximum(n_c, 1.0))
        out_ref[...] += jnp.full(
            (1, 1), jnp.where(n_c > 1.0, dist_term + reg_term, 0.0))

    out_ref[...] += jnp.full((1, 1), total)


def kernel(data, labels):
    labels = labels.astype(jnp.int32)
    partials = _seg_sums(data, labels)
    partials4 = partials.reshape(_NW, _B, _K, _D + 1)
    out = pl.pallas_call(
        _loss_body,
        grid=(_B, _NJ),
        in_specs=[
            pl.BlockSpec((1, _D, _BH, _W), lambda b, j: (b, 0, j, 0)),
            pl.BlockSpec((1, _BH, _W), lambda b, j: (b, j, 0)),
            pl.BlockSpec((_NW, 1, _K, _D + 1), lambda b, j: (0, b, 0, 0)),
        ],
        out_specs=pl.BlockSpec((1, 1), lambda b, j: (0, 0)),
        out_shape=jax.ShapeDtypeStruct((1, 1), jnp.float32),
    )(data, labels, partials4)
    return out[0, 0] / jnp.float32(_B)


# speedup vs baseline: 1.2232x; 1.1379x over previous
import functools

import jax
import jax.numpy as jnp
from jax import lax
from jax.experimental import pallas as pl
from jax.experimental.pallas import tpu as pltpu
from jax.experimental.pallas import tpu_sc as plsc

_B, _D, _H, _W, _K = 4, 16, 512, 512, 8
_DELTA_VAR = 1.0
_DELTA_DIST = 2.0
_SQRT_D = 4.0

_NW = 32
_RPW = _H // _NW
_CR = 4
_NCH = _RPW // _CR
_SEG = _K * (_D + 1)
_ACC = _B * _SEG
_GPC = _CR * _W // 16


def _seg_body(data_hbm, lab_hbm, out_hbm, lab_v, x_v, acc2, accf,
              sem0, sem1):
    wid = lax.axis_index("s") * 2 + lax.axis_index("c")
    row0 = wid * _RPW
    f32 = jnp.float32
    sems = (sem0, sem1)

    @plsc.parallel_loop(0, 16 * _ACC // 16, unroll=8)
    def _zero(i):
        acc2[pl.ds(i * 16, 16)] = jnp.zeros((16,), f32)

    def issue(n, buf):
        b = n // _NCH
        r = row0 + (n % _NCH) * _CR
        pltpu.async_copy(
            data_hbm.at[b, :, pl.ds(r, _CR), :], x_v.at[buf], sems[buf])
        pltpu.async_copy(
            lab_hbm.at[b, pl.ds(r, _CR), :], lab_v.at[buf], sems[buf])

    def wait_buf(buf):
        pltpu.make_async_copy(
            data_hbm.at[0, :, pl.ds(0, _CR), :], x_v.at[buf], sems[buf]).wait()
        pltpu.make_async_copy(
            lab_hbm.at[0, pl.ds(0, _CR), :], lab_v.at[buf], sems[buf]).wait()

    ntot = _B * _NCH
    ones16 = jnp.ones((16,), f32)
    laneoff = lax.iota(jnp.int32, 16)

    issue(0, 0)

    def outer(m, _):
        for buf in range(2):
            n = m * 2 + buf
            wait_buf(buf)

            @pl.when(n + 1 < ntot)
            def _prefetch():
                issue(n + 1, buf ^ 1)

            b_seg16 = (n // _NCH) * (_SEG * 16)

            @plsc.parallel_loop(0, _GPC, unroll=2)
            def _scat(g):
                r = g // (_W // 16)
                sl = pl.ds((g % (_W // 16)) * 16, 16)
                ix = (lab_v[buf, r, sl] * ((_D + 1) * 16)
                      + b_seg16 + laneoff)
                for d in range(_D):
                    plsc.addupdate_scatter(
                        acc2, [ix + d * 16], x_v[buf, d, r, sl])
                plsc.addupdate_scatter(acc2, [ix + _D * 16], ones16)

        return 0

    lax.fori_loop(0, ntot // 2, outer, 0)

    @plsc.parallel_loop(0, _ACC // 16, unroll=2)
    def _fold(j):
        s = jnp.zeros((16,), f32)
        base = j * 256 + lax.iota(jnp.int32, 16) * 16
        for m in range(16):
            s = s + plsc.load_gather(acc2, [base + m])
        accf[pl.ds(j * 16, 16)] = s

    pltpu.sync_copy(accf, out_hbm.at[wid])


@functools.lru_cache(maxsize=1)
def _seg_sums_kernel():
    return pl.kernel(
        _seg_body,
        mesh=plsc.VectorSubcoreMesh(core_axis_name="c", subcore_axis_name="s"),
        compiler_params=pltpu.CompilerParams(needs_layout_passes=False),
        out_type=jax.ShapeDtypeStruct((_NW, _ACC), jnp.float32),
        scratch_types=[
            pltpu.VMEM((2, _CR, _W), jnp.int32),
            pltpu.VMEM((2, _D, _CR, _W), jnp.float32),
            pltpu.VMEM((16 * _ACC,), jnp.float32),
            pltpu.VMEM((_ACC,), jnp.float32),
            pltpu.SemaphoreType.DMA,
            pltpu.SemaphoreType.DMA,
        ],
    )


def _seg_sums(data, labels):
    return _seg_sums_kernel()(data, labels)


_BH = 128
_NJ = _H // _BH


def _loss_body(data_ref, lab_ref, part_ref, out_ref):
    b = pl.program_id(0)
    j = pl.program_id(1)
    f32 = jnp.float32

    @pl.when((b == 0) & (j == 0))
    def _init():
        out_ref[...] = jnp.zeros((1, 1), f32)

    pm = jnp.sum(part_ref[...], axis=(0, 1))
    sums_t = pm[:, :_D]
    counts = pm[:, _D:_D + 1]
    centers_t = sums_t / jnp.maximum(counts, 1.0)
    present = counts > 0.0
    n_c = jnp.sum(present.astype(f32))
    cn2 = jnp.sum(centers_t * centers_t, axis=1, keepdims=True)
    ones_11 = jnp.ones((1, 1), f32)
    cn2_row = lax.dot_general(
        ones_11, cn2, (((1,), (1,)), ((), ())), preferred_element_type=f32)
    ones_1d = jnp.ones((1, _D), f32)
    dn_d = (((1,), (0,)), ((), ()))

    x = data_ref[0]
    lab = lab_ref[0]
    s3 = jnp.sum(x * x, axis=0)
    proj = lax.dot_general(
        centers_t, x, dn_d, preferred_element_type=f32)
    c0 = (lab & 1) == 1
    c1 = (lab & 2) == 2
    c2 = (lab & 4) == 4
    v0 = jnp.where(c0, cn2[1, 0], cn2[0, 0])
    v1 = jnp.where(c0, cn2[3, 0], cn2[2, 0])
    v2 = jnp.where(c0, cn2[5, 0], cn2[4, 0])
    v3 = jnp.where(c0, cn2[7, 0], cn2[6, 0])
    w0 = jnp.where(c1, v1, v0)
    w1 = jnp.where(c1, v3, v2)
    cnl = jnp.where(c2, w1, w0)
    p0 = jnp.where(c0, proj[1], proj[0])
    p1 = jnp.where(c0, proj[3], proj[2])
    p2 = jnp.where(c0, proj[5], proj[4])
    p3 = jnp.where(c0, proj[7], proj[6])
    q0 = jnp.where(c1, p1, p0)
    q1 = jnp.where(c1, p3, p2)
    t = jnp.where(c2, q1, q0)
    d2 = s3 + cnl - 2.0 * t
    dd = jnp.sqrt(jnp.maximum(d2, 0.0))
    h = jnp.maximum(dd - _DELTA_VAR, 0.0)
    var_sum = jnp.sum(h * h)

    total = jnp.where(n_c > 1.0, var_sum / jnp.maximum(n_c, 1.0), 0.0)

    @pl.when(j == 0)
    def _tiny_terms():
        g = lax.dot_general(
            centers_t, centers_t, (((1,), (1,)), ((), ())),
            preferred_element_type=f32)
        counts_row = lax.dot_general(
            ones_11, counts, (((1,), (1,)), ((), ())),
            preferred_element_type=f32)
        sq_c = cn2 + cn2_row - 2.0 * g
        ri = lax.broadcasted_iota(jnp.int32, (_K, _K), 0)
        ci = lax.broadcasted_iota(jnp.int32, (_K, _K), 1)
        pair_mask = (ri < ci) & present & (counts_row > 0.0)
        dist = jnp.sqrt(jnp.where(pair_mask, sq_c, 1.0))
        hc = jnp.maximum(2.0 * _DELTA_DIST - dist, 0.0) ** 2
        dist_sum = jnp.sum(jnp.where(pair_mask, hc, 0.0))
        dist_term = dist_sum / jnp.maximum(n_c * (n_c - 1.0), 1.0)

        reg_mask = present & (cn2 > 0.0)
        cn = jnp.sqrt(jnp.where(reg_mask, cn2, 1.0))
        reg_vals = jnp.maximum(cn - _SQRT_D, 0.0)
        reg_term = (jnp.sum(jnp.where(reg_mask, reg_vals, 0.0))
                    / jnp.maximum(n_c, 1.0))
        out_ref[...] += jnp.full(
            (1, 1), jnp.where(n_c > 1.0, dist_term + reg_term, 0.0))

    out_ref[...] += jnp.full((1, 1), total)


def kernel(data, labels):
    labels = labels.astype(jnp.int32)
    partials = _seg_sums(data, labels)
    partials4 = partials.reshape(_NW, _B, _K, _D + 1)
    out = pl.pallas_call(
        _loss_body,
        grid=(_B, _NJ),
        in_specs=[
            pl.BlockSpec((1, _D, _BH, _W), lambda b, j: (b, 0, j, 0)),
            pl.BlockSpec((1, _BH, _W), lambda b, j: (b, j, 0)),
            pl.BlockSpec((_NW, 1, _K, _D + 1), lambda b, j: (0, b, 0, 0)),
        ],
        out_specs=pl.BlockSpec((1, 1), lambda b, j: (0, 0)),
        out_shape=jax.ShapeDtypeStruct((1, 1), jnp.float32),
    )(data, labels, partials4)
    return out[0, 0] / jnp.float32(_B)
